# baseline (speedup 1.0000x reference)
import functools

import jax
import jax.numpy as jnp
from jax import lax
from jax.experimental import pallas as pl
from jax.experimental.pallas import tpu as pltpu
from jax.experimental.pallas import tpu_sc as plsc

DIM = 16
BATCH = 16384
NUM_CORES = 2
NUM_SUBCORES = 16
LANES = 16
NUM_WORKERS = NUM_CORES * NUM_SUBCORES
BPW = BATCH // NUM_WORKERS
CHUNK = 16
N_ROUNDS = BPW // CHUNK
N_GROUPS = CHUNK // LANES
BUF_COLS = CHUNK * LANES


@functools.partial(
    pl.kernel,
    out_type=jax.ShapeDtypeStruct((BATCH,), jnp.float32),
    mesh=plsc.VectorSubcoreMesh(core_axis_name="c", subcore_axis_name="s"),
    compiler_params=pltpu.CompilerParams(use_tc_tiling_on_sc=False,
                                         needs_layout_passes=False),
    scratch_types=[
        pltpu.VMEM((BPW,), jnp.int32),
        pltpu.VMEM((BPW,), jnp.int32),
        pltpu.VMEM((DIM, BUF_COLS), jnp.float32),
        pltpu.VMEM((DIM, BUF_COLS), jnp.float32),
        pltpu.VMEM((DIM, BUF_COLS), jnp.float32),
        pltpu.VMEM((DIM, BUF_COLS), jnp.float32),
        pltpu.VMEM((BPW,), jnp.float32),
        pltpu.SemaphoreType.DMA,
        pltpu.SemaphoreType.DMA,
        pltpu.SemaphoreType.DMA,
        pltpu.SemaphoreType.DMA,
    ],
)
def _mf_kernel(idx0_hbm, idx1_hbm, ut_hbm, vt_hbm, out_hbm,
               idx0_v, idx1_v, ub0, ub1, vb0, vb1, outv,
               sem_u0, sem_u1, sem_v0, sem_v1):
    wid = lax.axis_index("s") * NUM_CORES + lax.axis_index("c")
    base = wid * BPW

    pltpu.sync_copy(idx0_hbm.at[pl.ds(base, BPW)], idx0_v)
    pltpu.sync_copy(idx1_hbm.at[pl.ds(base, BPW)], idx1_v)

    lanes = lax.iota(jnp.int32, LANES)

    def fire_round(q, ub, vb, sem_u, sem_v):
        cbase = pl.multiple_of(q * CHUNK, CHUNK)
        for g in range(N_GROUPS):
            r0a = jnp.bitwise_and(idx0_v[pl.ds(cbase + g * LANES, LANES)], -16)
            r1a = jnp.bitwise_and(idx1_v[pl.ds(cbase + g * LANES, LANES)], -16)
            for j in range(LANES):
                e = g * LANES + j
                off_u = pl.multiple_of(r0a[j], LANES)
                off_v = pl.multiple_of(r1a[j], LANES)
                pltpu.async_copy(ut_hbm.at[:, pl.ds(off_u, LANES)],
                                 ub.at[:, pl.ds(e * LANES, LANES)], sem_u)
                pltpu.async_copy(vt_hbm.at[:, pl.ds(off_v, LANES)],
                                 vb.at[:, pl.ds(e * LANES, LANES)], sem_v)

    def drain_round(ub, vb, sem_u, sem_v):
        pltpu.make_async_copy(ut_hbm.at[:, pl.ds(0, BUF_COLS)], ub, sem_u).wait()
        pltpu.make_async_copy(vt_hbm.at[:, pl.ds(0, BUF_COLS)], vb, sem_v).wait()

    def compute_round(q, ub, vb):
        cbase = pl.multiple_of(q * CHUNK, CHUNK)
        for g in range(N_GROUPS):
            r0vec = idx0_v[pl.ds(cbase + g * LANES, LANES)]
            r1vec = idx1_v[pl.ds(cbase + g * LANES, LANES)]
            col_u = g * (LANES * LANES) + lanes * LANES + jnp.bitwise_and(r0vec, 15)
            col_v = g * (LANES * LANES) + lanes * LANES + jnp.bitwise_and(r1vec, 15)
            acc = jnp.zeros((LANES,), jnp.float32)
            for d in range(DIM):
                drow = jnp.full((LANES,), d, jnp.int32)
                acc = acc + (plsc.load_gather(ub, [drow, col_u]) *
                             plsc.load_gather(vb, [drow, col_v]))
            outv[pl.ds(cbase + g * LANES, LANES)] = acc

    fire_round(0, ub0, vb0, sem_u0, sem_v0)
    fire_round(1, ub1, vb1, sem_u1, sem_v1)

    def pipeline(q2, carry):
        q = pl.multiple_of(q2 * 2, 2)
        drain_round(ub0, vb0, sem_u0, sem_v0)
        compute_round(q, ub0, vb0)

        @pl.when(q + 2 < N_ROUNDS)
        def _():
            fire_round(q + 2, ub0, vb0, sem_u0, sem_v0)

        drain_round(ub1, vb1, sem_u1, sem_v1)
        compute_round(q + 1, ub1, vb1)

        @pl.when(q + 3 < N_ROUNDS)
        def _():
            fire_round(q + 3, ub1, vb1, sem_u1, sem_v1)

        return carry

    lax.fori_loop(0, N_ROUNDS // 2, pipeline, 0)

    pltpu.sync_copy(outv, out_hbm.at[pl.ds(base, BPW)])


def kernel(x, U, V):
    xi = x.astype(jnp.int32)
    return _mf_kernel(xi[:, 0], xi[:, 1], U.T, V.T)


# speedup vs baseline: 3.2228x; 3.2228x over previous
import functools

import jax
import jax.numpy as jnp
from jax import lax
from jax.experimental import pallas as pl
from jax.experimental.pallas import tpu as pltpu
from jax.experimental.pallas import tpu_sc as plsc

DIM = 16
BATCH = 16384
NUM_CORES = 2
NUM_SUBCORES = 16
LANES = 16
NUM_WORKERS = NUM_CORES * NUM_SUBCORES
BPW = BATCH // NUM_WORKERS
IDX_CHUNK = 128
N_CHUNKS = BPW // IDX_CHUNK


@functools.partial(
    pl.kernel,
    out_type=jax.ShapeDtypeStruct((BATCH,), jnp.float32),
    mesh=plsc.VectorSubcoreMesh(core_axis_name="c", subcore_axis_name="s"),
    compiler_params=pltpu.CompilerParams(use_tc_tiling_on_sc=False),
    scratch_types=[
        pltpu.VMEM((N_CHUNKS, IDX_CHUNK), jnp.int32),
        pltpu.VMEM((N_CHUNKS, IDX_CHUNK), jnp.int32),
        pltpu.VMEM((BPW, DIM), jnp.float32),
        pltpu.VMEM((BPW, DIM), jnp.float32),
        pltpu.VMEM((BPW,), jnp.float32),
        pltpu.SemaphoreType.DMA,
        pltpu.SemaphoreType.DMA,
    ],
)
def _mf_kernel(idx0_hbm, idx1_hbm, u_hbm, v_hbm, out_hbm,
               idx0_v, idx1_v, urows, vrows, outv, sem_u, sem_v):
    wid = lax.axis_index("s") * NUM_CORES + lax.axis_index("c")
    base = wid * BPW

    for j in range(N_CHUNKS):
        pltpu.sync_copy(idx0_hbm.at[pl.ds(base + j * IDX_CHUNK, IDX_CHUNK)],
                        idx0_v.at[j])
        pltpu.sync_copy(idx1_hbm.at[pl.ds(base + j * IDX_CHUNK, IDX_CHUNK)],
                        idx1_v.at[j])

    copies = []
    for j in range(N_CHUNKS):
        copies.append(pltpu.async_copy(
            u_hbm.at[idx0_v.at[j]],
            urows.at[pl.ds(j * IDX_CHUNK, IDX_CHUNK)], sem_u))
        copies.append(pltpu.async_copy(
            v_hbm.at[idx1_v.at[j]],
            vrows.at[pl.ds(j * IDX_CHUNK, IDX_CHUNK)], sem_v))
    for c in copies:
        c.wait()

    lanes = lax.iota(jnp.int32, LANES)
    perm8 = lanes ^ 8
    perm4 = lanes ^ 4
    perm2 = lanes ^ 2
    perm1 = lanes ^ 1

    def shuf(x, perm):
        return x.at[perm].get(mode="promise_in_bounds", unique_indices=True)

    def group(g, carry):
        gbase = pl.multiple_of(g * LANES, LANES)
        acc = jnp.zeros((LANES,), jnp.float32)
        for r in range(LANES):
            p = urows[gbase + r, :] * vrows[gbase + r, :]
            p = p + shuf(p, perm8)
            p = p + shuf(p, perm4)
            p = p + shuf(p, perm2)
            p = p + shuf(p, perm1)
            acc = jnp.where(lanes == r, p, acc)
        outv[pl.ds(gbase, LANES)] = acc
        return carry

    lax.fori_loop(0, BPW // LANES, group, 0)

    pltpu.sync_copy(outv, out_hbm.at[pl.ds(base, BPW)])


def kernel(x, U, V):
    xi = x.astype(jnp.int32)
    return _mf_kernel(xi[:, 0], xi[:, 1], U, V)
